# baseline (speedup 1.0000x reference)
import jax
import jax.numpy as jnp
from jax import lax
from jax.experimental import pallas as pl
from jax.experimental.pallas import tpu as pltpu
from jax.experimental.pallas import tpu_sc as plsc

N = 2_000_000
NC, NS, NW, L = 2, 16, 32, 16
BLK = 128
NBLK = N // BLK
B = 25
NCHUNKS = NBLK // B
MAXK = (NCHUNKS + NW - 1) // NW
DEPTH = 4
MAIN = (MAXK // DEPTH - 1) * DEPTH


def _sc_body(tgt_hbm, ip_hbm, out_hbm, shared_t, shared_i, stage, *sems):
    wid = lax.axis_index("s") * NC + lax.axis_index("c")
    sid = lax.axis_index("s")
    nk = (NCHUNKS + NW - 1 - wid) // NW

    def start(m, i):
        bb = (wid + m * NW) * B
        dst = shared_t.at[sid, i]
        pltpu.async_copy(tgt_hbm.at[pl.ds(bb, B), :, :],
                         dst.at[pl.ds(0, B), :, :], sems[i])
        pltpu.async_copy(tgt_hbm.at[pl.ds(NBLK + bb, B), :, :],
                         dst.at[pl.ds(B, B), :, :], sems[i])
        pltpu.async_copy(tgt_hbm.at[pl.ds(2 * NBLK + bb, B), :, :],
                         dst.at[pl.ds(2 * B, B), :, :], sems[i])
        pltpu.async_copy(ip_hbm.at[pl.ds(bb, B), :, :],
                         shared_i.at[sid, i], sems[i])

    def wait(i):
        dst = shared_t.at[sid, i]
        pltpu.make_async_copy(tgt_hbm.at[pl.ds(0, B), :, :],
                              dst.at[pl.ds(0, B), :, :], sems[i]).wait()
        pltpu.make_async_copy(tgt_hbm.at[pl.ds(0, B), :, :],
                              dst.at[pl.ds(B, B), :, :], sems[i]).wait()
        pltpu.make_async_copy(tgt_hbm.at[pl.ds(0, B), :, :],
                              dst.at[pl.ds(2 * B, B), :, :], sems[i]).wait()
        pltpu.make_async_copy(ip_hbm.at[pl.ds(0, B), :, :],
                              shared_i.at[sid, i], sems[i]).wait()

    for i in range(DEPTH):
        start(i, i)

    def ring_body(t4, acc):
        for i in range(DEPTH):
            m = t4 * DEPTH + i
            wait(i)

            @pl.when(m + DEPTH < nk)
            def _():
                start(m + DEPTH, i)

        return acc

    zero = jnp.zeros((L,), jnp.float32)
    acc = lax.fori_loop(0, MAIN // DEPTH, ring_body, zero)
    for m in range(MAIN, MAXK - 1):
        wait(m % DEPTH)

    @pl.when(MAXK - 1 < nk)
    def _():
        wait((MAXK - 1) % DEPTH)

    stage[pl.ds(0, L)] = acc
    stage[pl.ds(L, L)] = acc
    pltpu.sync_copy(stage, out_hbm.at[pl.ds(wid * (2 * L), 2 * L)])


@jax.jit
def _rpn_regr_loss(input_data, target):
    tgt_v = lax.reshape(target, (3 * NBLK, 1, BLK), dimensions=(0, 2, 1))
    ip_v = input_data[0].reshape(NBLK, BLK, 2).transpose(0, 2, 1)

    mesh = plsc.VectorSubcoreMesh(core_axis_name="c", subcore_axis_name="s")
    partials = pl.kernel(
        _sc_body,
        out_type=jax.ShapeDtypeStruct((NW * 2 * L,), jnp.float32),
        mesh=mesh,
        scratch_types=([pltpu.VMEM_SHARED((NS, DEPTH, 3 * B, 1, BLK), jnp.float32),
                        pltpu.VMEM_SHARED((NS, DEPTH, B, 2, BLK), jnp.float32)]
                       + [pltpu.VMEM((2 * L,), jnp.float32)]
                       + [pltpu.SemaphoreType.DMA] * DEPTH),
        compiler_params=pltpu.CompilerParams(needs_layout_passes=False),
    )(tgt_v, ip_v)
    p = partials.reshape(NW, 2, L)
    total = jnp.sum(p[:, 0, :])
    cnt = jnp.sum(p[:, 1, :])
    return jnp.where(cnt > 0, total / jnp.maximum(cnt, 1.0), 0.0) + 0.0 * total


def kernel(input_data, target):
    return _rpn_regr_loss(input_data.astype(jnp.float32),
                          target.astype(jnp.float32))


# speedup vs baseline: 1.1280x; 1.1280x over previous
import jax
import jax.numpy as jnp
from jax import lax
from jax.experimental import pallas as pl
from jax.experimental.pallas import tpu as pltpu
from jax.experimental.pallas import tpu_sc as plsc

N = 2_000_000
SIGMA = 9.0
T = 1.0 / SIGMA
HALF_SIGMA = 0.5 * SIGMA

NC = 2
NS = 16
NW = NC * NS
L = 16

BLK = 128
NBLK = N // BLK
B = 25
CH = B * BLK
NCHUNKS = NBLK // B
MAXK = (NCHUNKS + NW - 1) // NW
DEPTH = 4
MAIN = (MAXK // DEPTH - 1) * DEPTH


def _sc_body(tgt_hbm, ip_hbm, out_hbm, *scratch):
    bufs = [scratch[4 * i: 4 * i + 4] for i in range(DEPTH)]
    stage = scratch[4 * DEPTH]
    sems = scratch[4 * DEPTH + 1:]

    wid = lax.axis_index("s") * NC + lax.axis_index("c")
    nk = (NCHUNKS + NW - 1 - wid) // NW

    def start(m, i):
        cb, r0b, r1b, ib = bufs[i]
        bb = (wid + m * NW) * B
        pltpu.async_copy(tgt_hbm.at[pl.ds(bb, B), :, :], cb, sems[i])
        pltpu.async_copy(tgt_hbm.at[pl.ds(NBLK + bb, B), :, :], r0b, sems[i])
        pltpu.async_copy(tgt_hbm.at[pl.ds(2 * NBLK + bb, B), :, :], r1b, sems[i])
        pltpu.async_copy(ip_hbm.at[pl.ds(bb, B), :, :], ib, sems[i])

    def wait(i):
        cb, r0b, r1b, ib = bufs[i]
        pltpu.make_async_copy(tgt_hbm.at[pl.ds(0, B), :, :], cb, sems[i]).wait()
        pltpu.make_async_copy(tgt_hbm.at[pl.ds(0, B), :, :], r0b, sems[i]).wait()
        pltpu.make_async_copy(tgt_hbm.at[pl.ds(0, B), :, :], r1b, sems[i]).wait()
        pltpu.make_async_copy(ip_hbm.at[pl.ds(0, B), :, :], ib, sems[i]).wait()

    def compute(i, accs):
        cb, r0b, r1b, ib = bufs[i]

        @plsc.parallel_loop(0, B, 1, unroll=1, carry=accs)
        def block_body(b, carry2):
            a0, a1, c0, c1 = carry2
            for j in range(BLK // L):
                s = pl.ds(j * L, L)
                cls = cb[b, 0, s]
                r0 = r0b[b, 0, s]
                r1 = r1b[b, 0, s]
                p0 = ib[b, 0, s]
                p1 = ib[b, 1, s]
                d0 = jnp.abs(r0 - p0)
                d1 = jnp.abs(r1 - p1)
                m0 = jnp.minimum(d0, T)
                m1 = jnp.minimum(d1, T)
                l = (d0 - m0) + (d1 - m1) + HALF_SIGMA * (m0 * m0 + m1 * m1)
                if j % 2 == 0:
                    a0 = a0 + cls * l
                    c0 = c0 + cls
                else:
                    a1 = a1 + cls * l
                    c1 = c1 + cls
            return a0, a1, c0, c1

        return block_body

    for i in range(DEPTH):
        start(i, i)

    def ring_body(t4, accs):
        for i in range(DEPTH):
            m = t4 * DEPTH + i
            wait(i)
            accs = compute(i, accs)

            @pl.when(m + DEPTH < nk)
            def _():
                start(m + DEPTH, i)

        return accs

    zero = jnp.zeros((L,), jnp.float32)
    accs = lax.fori_loop(0, MAIN // DEPTH, ring_body, (zero, zero, zero, zero))

    for m in range(MAIN, MAXK - 1):
        wait(m % DEPTH)
        accs = compute(m % DEPTH, accs)

    def tail(accs_in):
        wait((MAXK - 1) % DEPTH)
        return compute((MAXK - 1) % DEPTH, accs_in)

    accs = lax.cond(MAXK - 1 < nk, tail, lambda accs_in: accs_in, accs)

    stage[pl.ds(0, L)] = accs[0] + accs[1]
    stage[pl.ds(L, L)] = accs[2] + accs[3]
    pltpu.sync_copy(stage, out_hbm.at[pl.ds(wid * (2 * L), 2 * L)])


@jax.jit
def _rpn_regr_loss(input_data, target):
    tgt_v = lax.reshape(target, (3 * NBLK, 1, BLK), dimensions=(0, 2, 1))
    ip_v = input_data[0].reshape(NBLK, BLK, 2).transpose(0, 2, 1)

    mesh = plsc.VectorSubcoreMesh(core_axis_name="c", subcore_axis_name="s")
    set_types = [
        pltpu.VMEM((B, 1, BLK), jnp.float32),
        pltpu.VMEM((B, 1, BLK), jnp.float32),
        pltpu.VMEM((B, 1, BLK), jnp.float32),
        pltpu.VMEM((B, 2, BLK), jnp.float32),
    ]
    partials = pl.kernel(
        _sc_body,
        out_type=jax.ShapeDtypeStruct((NW * 2 * L,), jnp.float32),
        mesh=mesh,
        scratch_types=(set_types * DEPTH
                       + [pltpu.VMEM((2 * L,), jnp.float32)]
                       + [pltpu.SemaphoreType.DMA] * DEPTH),
        compiler_params=pltpu.CompilerParams(needs_layout_passes=False),
    )(tgt_v, ip_v)
    p = partials.reshape(NW, 2, L)
    total = jnp.sum(p[:, 0, :])
    cnt = jnp.sum(p[:, 1, :])
    return jnp.where(cnt > 0, total / jnp.maximum(cnt, 1.0), 0.0)


def kernel(input_data, target):
    return _rpn_regr_loss(input_data.astype(jnp.float32),
                          target.astype(jnp.float32))
